# baseline (speedup 1.0000x reference)
import functools

import jax
import jax.numpy as jnp
from jax import lax
from jax.experimental import pallas as pl
from jax.experimental.pallas import tpu as pltpu
from jax.experimental.pallas import tpu_sc as plsc

LANES = 16


def kernel(x, token_table, pos_table):
    B, L = x.shape
    V, D = token_table.shape
    DG = D // LANES

    info = plsc.get_sparse_core_info()
    NC, NS = info.num_cores, info.num_subcores
    NW = NC * NS

    xT = jnp.swapaxes(x, 0, 1)
    tblT = jnp.swapaxes(token_table, 0, 1)
    posP = jnp.pad(jnp.swapaxes(pos_table, 0, 1), ((0, 0), (0, 256 - L)))
    tail2 = token_table[V - 128:, :].reshape(64, 2 * D)

    mesh = plsc.VectorSubcoreMesh(core_axis_name="c", subcore_axis_name="s")
    params = pltpu.CompilerParams(use_tc_tiling_on_sc=True, needs_layout_passes=False)

    NBLK = V // 128
    PER_W = NBLK // NW
    EXTRA = NBLK - PER_W * NW

    @functools.partial(
        pl.kernel,
        mesh=mesh,
        compiler_params=params,
        out_type=jax.ShapeDtypeStruct((V // 2, 2 * D), jnp.float32),
        scratch_types=[
            pltpu.VMEM((D, 128), jnp.float32),
            pltpu.VMEM((D, 128), jnp.float32),
        ],
    )
    def transpose_kernel(tblT_h, tail2_h, tbl2_h, in_v, stg_v):
        wid = lax.axis_index("s") * NC + lax.axis_index("c")
        iota = lax.iota(jnp.int32, LANES)
        half = lax.shift_right_logical(iota, 1)
        par64 = lax.mul(lax.rem(iota, 2), D)
        iota_k = [iota + k * LANES for k in range(8)]
        idx_l = [half + k * 8 for k in range(8)]
        start = PER_W * wid + jnp.minimum(wid, EXTRA)
        count = PER_W + jnp.where(wid < EXTRA, 1, 0)

        def blk_body(i, c):
            b = start + i
            pltpu.sync_copy(tblT_h.at[:, pl.ds(b * 128, 128)], in_v)

            def d_body(d, c2):
                splat_d = jnp.full_like(iota, d)
                idx_c = par64 + d
                for k in range(8):
                    val = plsc.load_gather(in_v, [splat_d, iota_k[k]])
                    plsc.store_scatter(stg_v, [idx_l[k], idx_c], val)
                return c2

            lax.fori_loop(0, D, d_body, 0)
            pltpu.sync_copy(stg_v, tbl2_h.at[pl.ds(b * 64, 64)])
            return c

        lax.fori_loop(0, count, blk_body, 0)

        @pl.when(wid == NW - 1)
        def _():
            pltpu.sync_copy(tail2_h, in_v)
            pltpu.sync_copy(in_v, tbl2_h.at[pl.ds(V // 2 - D, D)])

    tbl2 = transpose_kernel(tblT, tail2)

    NP8 = L // 8
    BC = B // NW

    @functools.partial(
        pl.kernel,
        mesh=mesh,
        compiler_params=params,
        out_type=jax.ShapeDtypeStruct((L, D, B), jnp.float32),
        scratch_types=[
            pltpu.VMEM((D, 256), jnp.float32),
            pltpu.VMEM((8, BC), jnp.int32),
            pltpu.VMEM((BC,), jnp.int32),
            pltpu.VMEM((BC,), jnp.int32),
            pltpu.VMEM((BC, 2 * D), jnp.float32),
            pltpu.VMEM((D, BC), jnp.float32),
            pltpu.SemaphoreType.DMA,
        ],
    )
    def emb_kernel(xT_h, tbl2_h, posP_h, out_h, pos_v, ids_v, idxq_v, m64_v,
                   g_v, stg_v, sem):
        wid = lax.axis_index("s") * NC + lax.axis_index("c")
        iota = lax.iota(jnp.int32, LANES)
        iota_k = [iota + k * LANES for k in range(BC // LANES)]
        iota_g = [iota + g * LANES for g in range(DG)]
        b0 = wid * BC
        pltpu.sync_copy(posP_h, pos_v)

        def p8_body(i, c):
            pltpu.sync_copy(xT_h.at[pl.ds(i * 8, 8), pl.ds(b0, BC)], ids_v)
            for p_loc in range(8):
                p_abs = i * 8 + p_loc
                splat_p = jnp.full_like(iota, p_abs)
                splat_pl = jnp.full_like(iota, p_loc)
                for k in range(BC // LANES):
                    ids = plsc.load_gather(ids_v, [splat_pl, iota_k[k]])
                    plsc.store_scatter(
                        idxq_v, [iota_k[k]], lax.shift_right_logical(ids, 1)
                    )
                    plsc.store_scatter(
                        m64_v, [iota_k[k]], lax.mul(lax.rem(ids, 2), D)
                    )
                pltpu.async_copy(tbl2_h.at[idxq_v], g_v, sem).wait()
                pvecs = [
                    plsc.load_gather(pos_v, [iota_g[g], splat_p])
                    for g in range(DG)
                ]

                def c_body(k, carry):
                    m64vec = plsc.load_gather(m64_v, [iota + k * LANES])
                    for j in range(LANES):
                        cc = k * LANES + j
                        m64 = m64vec[j]
                        idx_c = jnp.full_like(iota, cc)
                        splat_c = jnp.full_like(iota, cc)
                        for g in range(DG):
                            val = plsc.load_gather(
                                g_v, [splat_c, iota_g[g] + m64]
                            )
                            val = val + pvecs[g]
                            plsc.store_scatter(
                                stg_v, [iota_g[g], idx_c], val
                            )
                    return carry

                lax.fori_loop(0, BC // LANES, c_body, 0)
                pltpu.sync_copy(stg_v, out_h.at[p_abs, :, pl.ds(b0, BC)])
            return c

        lax.fori_loop(0, NP8, p8_body, 0)

    out = emb_kernel(xT, tbl2, posP)
    return jnp.transpose(out, (2, 0, 1))
